# baseline (speedup 1.0000x reference)
import functools

import jax
import jax.numpy as jnp
from jax import lax
from jax.experimental import pallas as pl
from jax.experimental.pallas import tpu as pltpu
from jax.experimental.pallas import tpu_sc as plsc

B = 16384
D = 32
NC = 2
NS = 16
NW = NC * NS
BPW = B // NW
CHUNK = 128
NCH = BPW // CHUNK

_sc_mesh = plsc.VectorSubcoreMesh(core_axis_name="c", subcore_axis_name="s")


@functools.partial(
    pl.kernel,
    mesh=_sc_mesh,
    compiler_params=pltpu.CompilerParams(use_tc_tiling_on_sc=False),
    out_type=jax.ShapeDtypeStruct((B, D), jnp.float32),
    scratch_types=[
        pltpu.VMEM((NCH, CHUNK), jnp.int32),
        pltpu.VMEM((BPW, D), jnp.float32),
        pltpu.SemaphoreType.DMA,
    ],
)
def _sc_gather_item(iid_hbm, itab_hbm, out_hbm, idx_v, rows_v, sem):
    wid = lax.axis_index("s") * NC + lax.axis_index("c")
    pltpu.sync_copy(iid_hbm.at[pl.ds(wid * NCH, NCH)], idx_v)
    copies = []
    for j in range(NCH):
        copies.append(pltpu.async_copy(
            itab_hbm.at[idx_v.at[j]],
            rows_v.at[pl.ds(j * CHUNK, CHUNK)], sem))
    for c in copies:
        c.wait()
    pltpu.sync_copy(rows_v, out_hbm.at[pl.ds(wid * BPW, BPW)])


BLK = 512
NSEM = 8


def _fused_body(ids_ref, tab_ref, xv_ref, w1a_ref, w1b_ref, b1_ref, w2_ref,
                b2_ref, w3_ref, b3_ref, out_ref, xu_a, xu_b, *sems):
    i = pl.program_id(0)
    nsteps = pl.num_programs(0)
    bufs = (xu_a, xu_b)

    def issue(step, buf):
        base = step * BLK
        for j in range(BLK):
            rid = ids_ref[base + j]
            pltpu.make_async_copy(
                tab_ref.at[pl.ds(rid, 1), :],
                buf.at[pl.ds(j, 1), :],
                sems[j % NSEM]).start()

    def drain(buf):
        for j in range(BLK):
            pltpu.make_async_copy(
                tab_ref.at[pl.ds(0, 1), :],
                buf.at[pl.ds(j, 1), :],
                sems[j % NSEM]).wait()

    @pl.when(i == 0)
    def _():
        issue(0, xu_a)

    cur = jax.lax.rem(i, 2)
    with jax.named_scope("drain_cur"):
        for b in range(2):
            @pl.when(cur == b)
            def _():
                drain(bufs[b])

    @pl.when(i + 1 < nsteps)
    def _():
        for b in range(2):
            @pl.when(cur == b)
            def _():
                issue(i + 1, bufs[1 - b])

    def mlp(xu):
        h = jnp.dot(xu, w1a_ref[...], preferred_element_type=jnp.float32)
        h = h + jnp.dot(xv_ref[...], w1b_ref[...],
                        preferred_element_type=jnp.float32)
        h = jnp.maximum(h + b1_ref[...], 0.0)
        h2 = jnp.dot(h, w2_ref[...], preferred_element_type=jnp.float32)
        h2 = jnp.maximum(h2 + b2_ref[...], 0.0)
        return jnp.sum(h2 * w3_ref[...], axis=1, keepdims=True) + b3_ref[...]

    xu = jnp.where((cur == 0), xu_a[...], xu_b[...])
    out_ref[...] = mlp(xu)


_w_spec = pl.BlockSpec((D, 64), lambda i, ids: (0, 0))


_fused = pl.pallas_call(
    _fused_body,
    grid_spec=pltpu.PrefetchScalarGridSpec(
        num_scalar_prefetch=1,
        grid=(B // BLK,),
        in_specs=[
            pl.BlockSpec(memory_space=pltpu.MemorySpace.HBM),
            pl.BlockSpec((BLK, D), lambda i, ids: (i, 0)),
            pl.BlockSpec((D, 64), lambda i, ids: (0, 0)),
            pl.BlockSpec((D, 64), lambda i, ids: (0, 0)),
            pl.BlockSpec((1, 64), lambda i, ids: (0, 0)),
            pl.BlockSpec((64, 32), lambda i, ids: (0, 0)),
            pl.BlockSpec((1, 32), lambda i, ids: (0, 0)),
            pl.BlockSpec((1, 32), lambda i, ids: (0, 0)),
            pl.BlockSpec((1, 1), lambda i, ids: (0, 0)),
        ],
        out_specs=pl.BlockSpec((BLK, 1), lambda i, ids: (i, 0)),
        scratch_shapes=[
            pltpu.VMEM((BLK, D), jnp.float32),
            pltpu.VMEM((BLK, D), jnp.float32),
        ] + [pltpu.SemaphoreType.DMA] * NSEM,
    ),
    out_shape=jax.ShapeDtypeStruct((B, 1), jnp.float32),
)


def kernel(user_ids, item_ids, user_table, item_table, W1, b1, W2, b2, W3, b3):
    uid = user_ids.astype(jnp.int32)
    iid = item_ids.astype(jnp.int32).reshape(B // CHUNK, CHUNK)
    irows = _sc_gather_item(iid, item_table)
    out = _fused(uid, user_table, irows, W1[:D], W1[D:], b1.reshape(1, 64),
                 W2, b2.reshape(1, 32), W3.reshape(1, 32), b3.reshape(1, 1))
    return out[:, 0]


# speedup vs baseline: 1.3102x; 1.3102x over previous
import functools

import jax
import jax.numpy as jnp
from jax import lax
from jax.experimental import pallas as pl
from jax.experimental.pallas import tpu as pltpu
from jax.experimental.pallas import tpu_sc as plsc

B = 16384
D = 32
NC = 2
NS = 16
NW = NC * NS
BPW = B // NW
CHUNK = 128
NCH = BPW // CHUNK

_sc_mesh = plsc.VectorSubcoreMesh(core_axis_name="c", subcore_axis_name="s")


@functools.partial(
    pl.kernel,
    mesh=_sc_mesh,
    out_type=jax.ShapeDtypeStruct((B, D), jnp.float32),
    scratch_types=[
        pltpu.VMEM((BPW,), jnp.int32),
        pltpu.SemaphoreType.DMA,
    ],
)
def _sc_gather_item(iid_hbm, itab_hbm, out_hbm, idx_v, sem):
    wid = lax.axis_index("s") * NC + lax.axis_index("c")
    base = wid * BPW
    pltpu.sync_copy(iid_hbm.at[pl.ds(base, BPW)], idx_v)

    def step(k, _):
        copies = []
        for sub in range(4):
            off = k * 64 + sub * 16
            vec = idx_v[pl.ds(off, 16)]
            for l in range(16):
                dst = base + off + l
                copies.append(pltpu.async_copy(
                    itab_hbm.at[pl.ds(vec[l], 1)],
                    out_hbm.at[pl.ds(dst, 1)], sem))
        for cp in copies:
            cp.wait()
        return _

    lax.fori_loop(0, BPW // 64, step, None)


BLK = 512
NSEM = 8


def _fused_body(ids_ref, tab_ref, xv_ref, w1a_ref, w1b_ref, b1_ref, w2_ref,
                b2_ref, w3_ref, b3_ref, out_ref, xu_a, xu_b, *sems):
    i = pl.program_id(0)
    nsteps = pl.num_programs(0)
    bufs = (xu_a, xu_b)

    def issue(step, buf):
        base = step * BLK
        for j in range(BLK):
            rid = ids_ref[base + j]
            pltpu.make_async_copy(
                tab_ref.at[pl.ds(rid, 1), :],
                buf.at[pl.ds(j, 1), :],
                sems[j % NSEM]).start()

    def drain(buf):
        for j in range(BLK):
            pltpu.make_async_copy(
                tab_ref.at[pl.ds(0, 1), :],
                buf.at[pl.ds(j, 1), :],
                sems[j % NSEM]).wait()

    @pl.when(i == 0)
    def _():
        issue(0, xu_a)

    cur = jax.lax.rem(i, 2)
    with jax.named_scope("drain_cur"):
        for b in range(2):
            @pl.when(cur == b)
            def _():
                drain(bufs[b])

    @pl.when(i + 1 < nsteps)
    def _():
        for b in range(2):
            @pl.when(cur == b)
            def _():
                issue(i + 1, bufs[1 - b])

    def mlp(xu):
        h = jnp.dot(xu, w1a_ref[...], preferred_element_type=jnp.float32)
        h = h + jnp.dot(xv_ref[...], w1b_ref[...],
                        preferred_element_type=jnp.float32)
        h = jnp.maximum(h + b1_ref[...], 0.0)
        h2 = jnp.dot(h, w2_ref[...], preferred_element_type=jnp.float32)
        h2 = jnp.maximum(h2 + b2_ref[...], 0.0)
        return jnp.sum(h2 * w3_ref[...], axis=1, keepdims=True) + b3_ref[...]

    xu = jnp.where((cur == 0), xu_a[...], xu_b[...])
    out_ref[...] = mlp(xu)


_w_spec = pl.BlockSpec((D, 64), lambda i, ids: (0, 0))


_fused = pl.pallas_call(
    _fused_body,
    grid_spec=pltpu.PrefetchScalarGridSpec(
        num_scalar_prefetch=1,
        grid=(B // BLK,),
        in_specs=[
            pl.BlockSpec(memory_space=pltpu.MemorySpace.HBM),
            pl.BlockSpec((BLK, D), lambda i, ids: (i, 0)),
            pl.BlockSpec((D, 64), lambda i, ids: (0, 0)),
            pl.BlockSpec((D, 64), lambda i, ids: (0, 0)),
            pl.BlockSpec((1, 64), lambda i, ids: (0, 0)),
            pl.BlockSpec((64, 32), lambda i, ids: (0, 0)),
            pl.BlockSpec((1, 32), lambda i, ids: (0, 0)),
            pl.BlockSpec((1, 32), lambda i, ids: (0, 0)),
            pl.BlockSpec((1, 1), lambda i, ids: (0, 0)),
        ],
        out_specs=pl.BlockSpec((BLK, 1), lambda i, ids: (i, 0)),
        scratch_shapes=[
            pltpu.VMEM((BLK, D), jnp.float32),
            pltpu.VMEM((BLK, D), jnp.float32),
        ] + [pltpu.SemaphoreType.DMA] * NSEM,
    ),
    out_shape=jax.ShapeDtypeStruct((B, 1), jnp.float32),
)


def kernel(user_ids, item_ids, user_table, item_table, W1, b1, W2, b2, W3, b3):
    uid = user_ids.astype(jnp.int32)
    iid = item_ids.astype(jnp.int32)
    irows = _sc_gather_item(iid, item_table)
    out = _fused(uid, user_table, irows, W1[:D], W1[D:], b1.reshape(1, 64),
                 W2, b2.reshape(1, 32), W3.reshape(1, 32), b3.reshape(1, 1))
    return out[:, 0]
